# baseline (speedup 1.0000x reference)
import functools

import jax
import jax.numpy as jnp
from jax import lax
from jax.experimental import pallas as pl
from jax.experimental.pallas import tpu as pltpu
from jax.experimental.pallas import tpu_sc as plsc

_LANES = 16
_CH = 2


def _splat(val, n=_LANES):
    return jnp.zeros((n,), jnp.int32) + val


def kernel(x, idxs):
    B, L, D = x.shape
    K = idxs.shape[1]
    OD = D * (1 + K)
    if idxs.dtype != jnp.int32:
        idxs = idxs.astype(jnp.int32)

    info = plsc.get_sparse_core_info()
    nc, ns = info.num_cores, info.num_subcores
    nw = nc * ns
    assert B % (nw * _CH) == 0
    nb = B // nw
    nch = nb // _CH

    mesh = plsc.VectorSubcoreMesh(core_axis_name="c", subcore_axis_name="s")

    @functools.partial(
        pl.kernel,
        out_type=jax.ShapeDtypeStruct((B, L, OD), jnp.float32),
        mesh=mesh,
        compiler_params=pltpu.CompilerParams(use_tc_tiling_on_sc=True),
        scratch_types=[
            pltpu.VMEM((nb, _LANES), jnp.int32),
            pltpu.VMEM((2, _CH, L, D), jnp.float32),
            pltpu.VMEM((2, _CH, L, OD), jnp.float32),
            pltpu.SemaphoreType.DMA((2,)),
            pltpu.SemaphoreType.DMA((2,)),
        ],
    )
    def body(x_hbm, idx_hbm, out_hbm, idx_v, xch, stage, sem_x, sem_o):
        wid = lax.axis_index("s") * nc + lax.axis_index("c")
        b0 = wid * nb
        lane_lo = lax.iota(jnp.int32, _LANES)
        pltpu.sync_copy(idx_hbm.at[pl.ds(b0, nb)], idx_v)

        def x_copy(c, slot):
            return pltpu.make_async_copy(
                x_hbm.at[pl.ds(b0 + c * _CH, _CH)], xch.at[slot], sem_x.at[slot]
            )

        def out_copy(c, st):
            return pltpu.make_async_copy(
                stage.at[st],
                out_hbm.at[pl.ds(b0 + c * _CH, _CH)],
                sem_o.at[st],
            )

        x_copy(0, 0).start()

        def chunk_pair(i, carry):
            c2 = i * 2
            for h in (0, 1):
                c = c2 + h

                @pl.when(c + 1 < nch)
                def _():
                    x_copy(c + 1, 1 - h).start()

                x_copy(c, h).wait()

                @pl.when(c >= 2)
                def _():
                    out_copy(c - 2, h).wait()

                for j_local in range(_CH):
                    jb = c * _CH + j_local
                    iv = idx_v[jb, pl.ds(0, _LANES)]
                    gv = []
                    for kk in range(K):
                        idx_s = iv[kk]
                        for half in range(D // _LANES):
                            gv.append(
                                xch[h, j_local, idx_s, pl.ds(half * _LANES, _LANES)]
                            )
                    for l in range(L):
                        for half in range(D // _LANES):
                            stage[h, j_local, l, pl.ds(half * _LANES, _LANES)] = xch[
                                h, j_local, l, pl.ds(half * _LANES, _LANES)
                            ]
                        for ci in range(len(gv)):
                            stage[h, j_local, l, pl.ds(D + ci * _LANES, _LANES)] = gv[
                                ci
                            ]

                out_copy(c, h).start()
            return carry

        lax.fori_loop(0, nch // 2, chunk_pair, 0)

        out_copy(nch - 2, 0).wait()
        out_copy(nch - 1, 1).wait()

    idxs16 = jnp.pad(idxs, ((0, 0), (0, _LANES - K)))
    return body(x, idxs16)


# speedup vs baseline: 1.0013x; 1.0013x over previous
import functools

import jax
import jax.numpy as jnp
from jax import lax
from jax.experimental import pallas as pl
from jax.experimental.pallas import tpu as pltpu
from jax.experimental.pallas import tpu_sc as plsc

_LANES = 16
_CH = 2


def _splat(val, n=_LANES):
    return jnp.zeros((n,), jnp.int32) + val


def kernel(x, idxs):
    B, L, D = x.shape
    K = idxs.shape[1]
    OD = D * (1 + K)
    if idxs.dtype != jnp.int32:
        idxs = idxs.astype(jnp.int32)

    info = plsc.get_sparse_core_info()
    nc, ns = info.num_cores, info.num_subcores
    nw = nc * ns
    assert B % (nw * _CH) == 0
    nb = B // nw
    nch = nb // _CH

    mesh = plsc.VectorSubcoreMesh(core_axis_name="c", subcore_axis_name="s")

    @functools.partial(
        pl.kernel,
        out_type=jax.ShapeDtypeStruct((B, L, OD), jnp.float32),
        mesh=mesh,
        compiler_params=pltpu.CompilerParams(
            use_tc_tiling_on_sc=True, skip_device_barrier=True
        ),
        scratch_types=[
            pltpu.VMEM((nb, _LANES), jnp.int32),
            pltpu.VMEM((2, _CH, L, D), jnp.float32),
            pltpu.VMEM((2, _CH, L, OD), jnp.float32),
            pltpu.SemaphoreType.DMA((2,)),
            pltpu.SemaphoreType.DMA((2,)),
        ],
    )
    def body(x_hbm, idx_hbm, out_hbm, idx_v, xch, stage, sem_x, sem_o):
        wid = lax.axis_index("s") * nc + lax.axis_index("c")
        b0 = wid * nb
        lane_lo = lax.iota(jnp.int32, _LANES)
        pltpu.sync_copy(idx_hbm.at[pl.ds(b0, nb)], idx_v)

        def x_copy(c, slot):
            return pltpu.make_async_copy(
                x_hbm.at[pl.ds(b0 + c * _CH, _CH)], xch.at[slot], sem_x.at[slot]
            )

        def out_copy(c, st):
            return pltpu.make_async_copy(
                stage.at[st],
                out_hbm.at[pl.ds(b0 + c * _CH, _CH)],
                sem_o.at[st],
            )

        x_copy(0, 0).start()

        def chunk_pair(i, carry):
            c2 = i * 2
            for h in (0, 1):
                c = c2 + h

                @pl.when(c + 1 < nch)
                def _():
                    x_copy(c + 1, 1 - h).start()

                x_copy(c, h).wait()

                @pl.when(c >= 2)
                def _():
                    out_copy(c - 2, h).wait()

                for j_local in range(_CH):
                    jb = c * _CH + j_local
                    iv = idx_v[jb, pl.ds(0, _LANES)]
                    gv = []
                    for kk in range(K):
                        idx_s = iv[kk]
                        for half in range(D // _LANES):
                            gv.append(
                                xch[h, j_local, idx_s, pl.ds(half * _LANES, _LANES)]
                            )
                    for l in range(L):
                        for half in range(D // _LANES):
                            stage[h, j_local, l, pl.ds(half * _LANES, _LANES)] = xch[
                                h, j_local, l, pl.ds(half * _LANES, _LANES)
                            ]
                        for ci in range(len(gv)):
                            stage[h, j_local, l, pl.ds(D + ci * _LANES, _LANES)] = gv[
                                ci
                            ]

                out_copy(c, h).start()
            return carry

        lax.fori_loop(0, nch // 2, chunk_pair, 0)

        out_copy(nch - 2, 0).wait()
        out_copy(nch - 1, 1).wait()

    idxs16 = jnp.pad(idxs, ((0, 0), (0, _LANES - K)))
    return body(x, idxs16)


# speedup vs baseline: 1.0101x; 1.0088x over previous
import functools

import jax
import jax.numpy as jnp
from jax import lax
from jax.experimental import pallas as pl
from jax.experimental.pallas import tpu as pltpu
from jax.experimental.pallas import tpu_sc as plsc

_LANES = 16
_CH = 2


def kernel(x, idxs):
    B, L, D = x.shape
    K = idxs.shape[1]
    OD = D * (1 + K)
    if idxs.dtype != jnp.int32:
        idxs = idxs.astype(jnp.int32)

    info = plsc.get_sparse_core_info()
    nc, ns = info.num_cores, info.num_subcores
    nw = nc * ns
    assert B % (nw * _CH) == 0
    nb = B // nw
    nch = nb // _CH

    mesh = plsc.VectorSubcoreMesh(core_axis_name="c", subcore_axis_name="s")

    @functools.partial(
        pl.kernel,
        out_type=jax.ShapeDtypeStruct((B, L, OD), jnp.float32),
        mesh=mesh,
        compiler_params=pltpu.CompilerParams(use_tc_tiling_on_sc=True),
        scratch_types=[
            pltpu.VMEM((nb, _LANES), jnp.int32),
            pltpu.VMEM((2, _CH, L, D), jnp.float32),
            pltpu.VMEM((2, _CH, L, OD), jnp.float32),
            pltpu.SemaphoreType.DMA((2,)),
            pltpu.SemaphoreType.DMA((2,)),
        ],
    )
    def body(x_hbm, idx_hbm, out_hbm, idx_v, xch, stage, sem_x, sem_o):
        wid = lax.axis_index("s") * nc + lax.axis_index("c")
        b0 = wid * nb
        pltpu.sync_copy(idx_hbm.at[pl.ds(b0, nb)], idx_v)

        def x_copy(c, slot):
            return pltpu.make_async_copy(
                x_hbm.at[pl.ds(b0 + c * _CH, _CH)], xch.at[slot], sem_x.at[slot]
            )

        def out_copy(c, st):
            return pltpu.make_async_copy(
                stage.at[st],
                out_hbm.at[pl.ds(b0 + c * _CH, _CH)],
                sem_o.at[st],
            )

        x_copy(0, 0).start()

        def chunk_pair(i, carry):
            c2 = i * 2
            for h in (0, 1):
                c = c2 + h

                @pl.when(c + 1 < nch)
                def _():
                    x_copy(c + 1, 1 - h).start()

                x_copy(c, h).wait()

                @pl.when(c >= 2)
                def _():
                    out_copy(c - 2, h).wait()

                for j_local in range(_CH):
                    jb = c * _CH + j_local
                    iv = idx_v[jb, pl.ds(0, _LANES)]
                    gv = []
                    for kk in range(K):
                        idx_s = iv[kk]
                        for half in range(D // _LANES):
                            gv.append(
                                xch[h, j_local, idx_s, pl.ds(half * _LANES, _LANES)]
                            )
                    for l in range(L):
                        for half in range(D // _LANES):
                            stage[h, j_local, l, pl.ds(half * _LANES, _LANES)] = xch[
                                h, j_local, l, pl.ds(half * _LANES, _LANES)
                            ]
                        for ci in range(len(gv)):
                            stage[h, j_local, l, pl.ds(D + ci * _LANES, _LANES)] = gv[
                                ci
                            ]

                out_copy(c, h).start()
            return carry

        lax.fori_loop(0, nch // 2, chunk_pair, 0)

        out_copy(nch - 2, 0).wait()
        out_copy(nch - 1, 1).wait()

    idxs16 = jnp.pad(idxs, ((0, 0), (0, _LANES - K)))
    return body(x, idxs16)
